# baseline (speedup 1.0000x reference)
import jax
import jax.numpy as jnp
from jax import lax
from jax.experimental import pallas as pl
from jax.experimental.pallas import tpu as pltpu
from jax.experimental.pallas import tpu_sc as plsc

NUM_CORES = 2
NUM_SUBCORES = 16
NW = NUM_CORES * NUM_SUBCORES

V = 100
D = 128
L = 16
CHUNK = 128
GROUPS = CHUNK // L
CPW = 25
NBUF = 5
ROWS_PER_WORKER = CPW * CHUNK
B_PAD = NW * ROWS_PER_WORKER
CW = CHUNK * D


def _emb_body(idx_hbm, table_hbm, out_hbm, table_v, idx_v, rows_v, sem_s):
    wid = lax.axis_index("s") * NUM_CORES + lax.axis_index("c")
    pltpu.sync_copy(table_hbm, table_v)
    pltpu.sync_copy(idx_hbm.at[wid], idx_v)
    lanes_d = lax.iota(jnp.int32, L) * D

    def do_chunk(c, b):
        @pl.when(c >= NBUF)
        def _():
            pltpu.make_async_copy(rows_v.at[b], out_hbm.at[0], sem_s).wait()

        buf = rows_v.at[b].at[0]
        for g in range(GROUPS):
            x = idx_v[0, pl.ds(c * CHUNK + g * L, L)]
            xw = x * D
            dbase = lanes_d + (g * L * D)

            def col_body(col, carry):
                v = plsc.load_gather(table_v, [xw + col])
                plsc.store_scatter(buf, [dbase + col], v)
                return carry

            lax.fori_loop(0, D, col_body, 0, unroll=8)
        pltpu.async_copy(rows_v.at[b], out_hbm.at[wid * CPW + c], sem_s)

    def outer(o, carry):
        for b in range(NBUF):
            do_chunk(o * NBUF + b, b)
        return carry

    lax.fori_loop(0, CPW // NBUF, outer, 0)
    for b in range(NBUF):
        pltpu.make_async_copy(rows_v.at[b], out_hbm.at[0], sem_s).wait()


@jax.jit
def _emb(idx3, table_flat):
    mesh = plsc.VectorSubcoreMesh(core_axis_name="c", subcore_axis_name="s")
    return pl.kernel(
        _emb_body,
        out_type=jax.ShapeDtypeStruct((NW * CPW, 1, CW), jnp.float32),
        mesh=mesh,
        compiler_params=pltpu.CompilerParams(needs_layout_passes=False),
        scratch_types=[
            pltpu.VMEM((V * D,), jnp.float32),
            pltpu.VMEM((1, ROWS_PER_WORKER), jnp.int32),
            pltpu.VMEM((NBUF, 1, CW), jnp.float32),
            pltpu.SemaphoreType.DMA,
        ],
    )(idx3, table_flat)


def kernel(x, table):
    n = x.shape[0]
    x_pad = jnp.pad(x.astype(jnp.int32), (0, B_PAD - n))
    idx3 = x_pad.reshape(NW, 1, ROWS_PER_WORKER)
    out = _emb(idx3, table.reshape(-1))
    return out.reshape(B_PAD, D)[:n]


# speedup vs baseline: 1.6905x; 1.6905x over previous
import jax
import jax.numpy as jnp
from jax import lax
from jax.experimental import pallas as pl
from jax.experimental.pallas import tpu as pltpu
from jax.experimental.pallas import tpu_sc as plsc

NUM_CORES = 2
NUM_SUBCORES = 16
NW = NUM_CORES * NUM_SUBCORES

V = 100
D = 128
L = 16
CHUNK = 128
GROUPS = CHUNK // L
CPW = 25
NBUF = 5
ROWS_PER_WORKER = CPW * CHUNK
B_PAD = NW * ROWS_PER_WORKER
CW = CHUNK * D


def _emb_body(idx_hbm, table_hbm, out_hbm, table_v, idx_v, rows_v, sem_s):
    wid = lax.axis_index("s") * NUM_CORES + lax.axis_index("c")
    pltpu.sync_copy(table_hbm, table_v)
    pltpu.sync_copy(idx_hbm.at[wid], idx_v)
    lanes_d = lax.iota(jnp.int32, L) * D

    def do_chunk(c, b):
        @pl.when(c >= NBUF)
        def _():
            pltpu.make_async_copy(rows_v.at[b], out_hbm.at[0], sem_s).wait()

        buf = rows_v.at[b].at[0]
        xws = []
        dbs = []
        for g in range(GROUPS):
            x = idx_v[0, pl.ds(c * CHUNK + g * L, L)]
            xws.append(x * D)
            dbs.append(lanes_d + g * L * D)

        @plsc.parallel_loop(0, D, 1, unroll=4)
        def _(col):
            vs = [plsc.load_gather(table_v, [xws[g] + col]) for g in range(GROUPS)]
            for g in range(GROUPS):
                plsc.store_scatter(buf, [dbs[g] + col], vs[g])
        pltpu.async_copy(rows_v.at[b], out_hbm.at[wid * CPW + c], sem_s)

    def outer(o, carry):
        for b in range(NBUF):
            do_chunk(o * NBUF + b, b)
        return carry

    lax.fori_loop(0, CPW // NBUF, outer, 0)
    for b in range(NBUF):
        pltpu.make_async_copy(rows_v.at[b], out_hbm.at[0], sem_s).wait()


@jax.jit
def _emb(idx3, table_flat):
    mesh = plsc.VectorSubcoreMesh(core_axis_name="c", subcore_axis_name="s")
    return pl.kernel(
        _emb_body,
        out_type=jax.ShapeDtypeStruct((NW * CPW, 1, CW), jnp.float32),
        mesh=mesh,
        compiler_params=pltpu.CompilerParams(needs_layout_passes=False),
        scratch_types=[
            pltpu.VMEM((V * D,), jnp.float32),
            pltpu.VMEM((1, ROWS_PER_WORKER), jnp.int32),
            pltpu.VMEM((NBUF, 1, CW), jnp.float32),
            pltpu.SemaphoreType.DMA,
        ],
    )(idx3, table_flat)


def kernel(x, table):
    n = x.shape[0]
    x_pad = jnp.pad(x.astype(jnp.int32), (0, B_PAD - n))
    idx3 = x_pad.reshape(NW, 1, ROWS_PER_WORKER)
    out = _emb(idx3, table.reshape(-1))
    return out.reshape(B_PAD, D)[:n]


# speedup vs baseline: 4.3943x; 2.5994x over previous
import jax
import jax.numpy as jnp
from jax import lax
from jax.experimental import pallas as pl
from jax.experimental.pallas import tpu as pltpu
from jax.experimental.pallas import tpu_sc as plsc

NUM_CORES = 2
NUM_SUBCORES = 16
NW = NUM_CORES * NUM_SUBCORES

V = 100
D = 128
L = 16
CHUNK = 128
GROUPS = CHUNK // L
CPW = 25
NBUF = 5
ROWS_PER_WORKER = CPW * CHUNK
B_PAD = NW * ROWS_PER_WORKER
CW = CHUNK * D


def _emb_body(idx_hbm, table_hbm, out_hbm, table_v, idx_v, rows_v, sem_s):
    wid = lax.axis_index("s") * NUM_CORES + lax.axis_index("c")
    pltpu.sync_copy(table_hbm, table_v)
    pltpu.sync_copy(idx_hbm.at[wid], idx_v)
    lanes_d = lax.iota(jnp.int32, L) * D

    def do_chunk(c, b):
        @pl.when(c >= NBUF)
        def _():
            pltpu.make_async_copy(rows_v.at[b], out_hbm.at[0], sem_s).wait()

        buf = rows_v.at[b].at[0]

        @plsc.parallel_loop(0, GROUPS, 1, unroll=1)
        def _(g):
            x = idx_v[0, pl.ds(c * CHUNK + g * L, L)]
            srcs = x * D
            for l in range(L):
                src = srcs[l]
                dst = g * L * D + l * D
                for j in range(D // L):
                    buf[pl.ds(dst + j * L, L)] = table_v[pl.ds(src + j * L, L)]
        pltpu.async_copy(rows_v.at[b], out_hbm.at[wid * CPW + c], sem_s)

    def outer(o, carry):
        for b in range(NBUF):
            do_chunk(o * NBUF + b, b)
        return carry

    lax.fori_loop(0, CPW // NBUF, outer, 0)
    for b in range(NBUF):
        pltpu.make_async_copy(rows_v.at[b], out_hbm.at[0], sem_s).wait()


@jax.jit
def _emb(idx3, table_flat):
    mesh = plsc.VectorSubcoreMesh(core_axis_name="c", subcore_axis_name="s")
    return pl.kernel(
        _emb_body,
        out_type=jax.ShapeDtypeStruct((NW * CPW, 1, CW), jnp.float32),
        mesh=mesh,
        compiler_params=pltpu.CompilerParams(needs_layout_passes=False),
        scratch_types=[
            pltpu.VMEM((V * D,), jnp.float32),
            pltpu.VMEM((1, ROWS_PER_WORKER), jnp.int32),
            pltpu.VMEM((NBUF, 1, CW), jnp.float32),
            pltpu.SemaphoreType.DMA,
        ],
    )(idx3, table_flat)


def kernel(x, table):
    n = x.shape[0]
    x_pad = jnp.pad(x.astype(jnp.int32), (0, B_PAD - n))
    idx3 = x_pad.reshape(NW, 1, ROWS_PER_WORKER)
    out = _emb(idx3, table.reshape(-1))
    return out.reshape(B_PAD, D)[:n]


# speedup vs baseline: 10.4879x; 2.3867x over previous
import jax
import jax.numpy as jnp
from jax import lax
from jax.experimental import pallas as pl
from jax.experimental.pallas import tpu as pltpu
from jax.experimental.pallas import tpu_sc as plsc

NUM_CORES = 2
NUM_SUBCORES = 16
NW = NUM_CORES * NUM_SUBCORES

V = 100
D = 128
L = 16
N_ROWS = 100000
CHUNK = 128
GROUPS = CHUNK // L
CW = CHUNK * D
NCHUNKS = 782
CPW = 25
NBUF = 5


def _emb_body(idx_hbm, table_hbm, out_hbm, table_v, idx_v, rows_v, sem_s):
    wid = lax.axis_index("s") * NUM_CORES + lax.axis_index("c")
    pltpu.sync_copy(table_hbm, table_v)
    pltpu.sync_copy(idx_hbm.at[wid], idx_v)

    def drain(b):
        pltpu.make_async_copy(
            rows_v.at[b].at[0], out_hbm.at[pl.ds(0, CW)], sem_s
        ).wait()

    def do_chunk(t, b):
        @pl.when(t >= NBUF)
        def _():
            drain(b)

        k = wid + NW * t

        @pl.when(k < NCHUNKS)
        def _():
            buf = rows_v.at[b].at[0]

            @plsc.parallel_loop(0, GROUPS, 1, unroll=1)
            def _(g):
                xvec = idx_v[0, pl.ds(t * CHUNK + g * L, L)]
                srcs = xvec * D
                for l in range(L):
                    src = srcs[l]
                    dst = g * L * D + l * D
                    for j in range(D // L):
                        buf[pl.ds(dst + j * L, L)] = table_v[pl.ds(src + j * L, L)]

            off = jnp.minimum(k * CHUNK, N_ROWS - CHUNK)
            pltpu.async_copy(
                rows_v.at[b].at[0], out_hbm.at[pl.ds(off * D, CW)], sem_s
            )

    def outer(t, carry):
        do_chunk(t, t % NBUF)
        return carry

    lax.fori_loop(0, CPW, outer, 0)
    for t in range(CPW - NBUF, CPW):
        b = t % NBUF
        if t == CPW - 1:
            @pl.when(wid + NW * t < NCHUNKS)
            def _():
                drain(b)
        else:
            drain(b)


@jax.jit
def _emb(idx3, table_flat):
    mesh = plsc.VectorSubcoreMesh(core_axis_name="c", subcore_axis_name="s")
    return pl.kernel(
        _emb_body,
        out_type=jax.ShapeDtypeStruct((N_ROWS * D,), jnp.float32),
        mesh=mesh,
        compiler_params=pltpu.CompilerParams(needs_layout_passes=False),
        scratch_types=[
            pltpu.VMEM((V * D,), jnp.float32),
            pltpu.VMEM((1, CPW * CHUNK), jnp.int32),
            pltpu.VMEM((NBUF, 1, CW), jnp.float32),
            pltpu.SemaphoreType.DMA,
        ],
    )(idx3, table_flat)


def kernel(x, table):
    xi = x.astype(jnp.int32)
    main = xi[: (NCHUNKS - 1) * CHUNK].reshape(NCHUNKS - 1, CHUNK)
    tail = xi[N_ROWS - CHUNK:].reshape(1, CHUNK)
    pad = jnp.zeros((NW * CPW - NCHUNKS, CHUNK), jnp.int32)
    chunks = jnp.concatenate([main, tail, pad], axis=0)
    idx3 = (
        chunks.reshape(CPW, NW, CHUNK)
        .transpose(1, 0, 2)
        .reshape(NW, 1, CPW * CHUNK)
    )
    out = _emb(idx3, table.reshape(-1))
    return out.reshape(N_ROWS, D)
